# Initial kernel scaffold; baseline (speedup 1.0000x reference)
import jax
import jax.numpy as jnp
from jax.experimental import pallas as pl


def kernel(input, W0, W1, W2, Wm, b):
    raise NotImplementedError("write your pallas kernel here")


# speedup vs baseline: 3.1309x; 3.1309x over previous
import functools

import jax
import jax.numpy as jnp
from jax import lax
from jax.experimental import pallas as pl
from jax.experimental.pallas import tpu as pltpu
from jax.experimental.pallas import tpu_sc as plsc

DIM = 64
LANES = 16
NC, NS = 2, 16
NW = NC * NS
SUB = 128
SUBS_PER_CHUNK = 2
CHUNK = SUB * SUBS_PER_CHUNK

PROJ_BLK = 1000


def _project_kernel(w0, w1, w2, m0, m1, m2, bb, pa, pb):
    c0 = jnp.dot(w0[...], m0[...], preferred_element_type=jnp.float32)
    c1 = jnp.dot(w1[...], m1[...], preferred_element_type=jnp.float32)
    c2 = jnp.dot(w2[...], m2[...], preferred_element_type=jnp.float32)
    third = bb[...]
    pa[...] = jnp.concatenate([c0 + third, c1 + third], axis=1)
    pb[...] = jnp.concatenate([c2 + third, c2 + third], axis=1)


def _project(W0, W1, W2, M0, M1, M2, bb):
    vocab = W0.shape[0]
    assert vocab % PROJ_BLK == 0
    nblk = vocab // PROJ_BLK
    row_spec = pl.BlockSpec((PROJ_BLK, DIM), lambda i: (i, 0))
    out_spec = pl.BlockSpec((PROJ_BLK, 2 * DIM), lambda i: (i, 0))
    full_spec = pl.BlockSpec((DIM, DIM), lambda i: (0, 0))
    bias_spec = pl.BlockSpec((1, DIM), lambda i: (0, 0))
    return pl.pallas_call(
        _project_kernel,
        grid=(nblk,),
        in_specs=[row_spec, row_spec, row_spec,
                  full_spec, full_spec, full_spec, bias_spec],
        out_specs=[out_spec, out_spec],
        out_shape=[jax.ShapeDtypeStruct((vocab, 2 * DIM), jnp.float32)] * 2,
    )(W0, W1, W2, M0, M1, M2, bb)


def _make_gather_sum(n_tokens):
    n_per_w = n_tokens // NW
    n_chunks = n_per_w // CHUNK
    assert n_per_w % CHUNK == 0
    mesh = plsc.VectorSubcoreMesh(core_axis_name="c", subcore_axis_name="s")

    @functools.partial(
        pl.kernel,
        mesh=mesh,
        out_type=jax.ShapeDtypeStruct((n_tokens // 2, 2 * DIM), jnp.float32),
        scratch_types=[
            pltpu.VMEM((CHUNK,), jnp.int32),
            pltpu.VMEM((CHUNK,), jnp.int32),
            pltpu.VMEM((CHUNK,), jnp.int32),
            pltpu.VMEM((CHUNK, 2 * DIM), jnp.float32),
            pltpu.VMEM((CHUNK, 2 * DIM), jnp.float32),
            pltpu.VMEM((CHUNK, 2 * DIM), jnp.float32),
            pltpu.VMEM((CHUNK // 2, 2 * DIM), jnp.float32),
            pltpu.SemaphoreType.DMA,
        ],
    )
    def gather_sum(pa_hbm, pb_hbm, i0_hbm, i1_hbm, i2_hbm,
                   out_hbm, i0_v, i1_v, i2_v, r0_v, r1_v, r2_v, o_v, sem):
        wid = lax.axis_index("s") * NC + lax.axis_index("c")
        base = wid * n_per_w

        def chunk_body(ch, _):
            tok0 = pl.multiple_of(base + ch * CHUNK, CHUNK)
            toks = pl.ds(tok0, CHUNK)
            pltpu.sync_copy(i0_hbm.at[toks], i0_v)
            pltpu.sync_copy(i1_hbm.at[toks], i1_v)
            pltpu.sync_copy(i2_hbm.at[toks], i2_v)
            copies = []
            for j in range(SUBS_PER_CHUNK):
                sub = pl.ds(j * SUB, SUB)
                copies.append(pltpu.async_copy(
                    pa_hbm.at[i0_v.at[sub]], r0_v.at[sub], sem))
                copies.append(pltpu.async_copy(
                    pa_hbm.at[i1_v.at[sub]], r1_v.at[sub], sem))
                copies.append(pltpu.async_copy(
                    pb_hbm.at[i2_v.at[sub]], r2_v.at[sub], sem))
            for cp in copies:
                cp.wait()

            def pair_body(u, _):
                for r in range(2):
                    t = 2 * u + r
                    for k in range(DIM // LANES):
                        src = pl.ds(k * LANES, LANES)
                        hi = pl.ds(DIM + k * LANES, LANES)
                        v = r0_v[t, src] + r1_v[t, hi] + r2_v[t, src]
                        o_v[u, pl.ds(r * DIM + k * LANES, LANES)] = (
                            jnp.maximum(v, 0.0))
                return 0

            lax.fori_loop(0, CHUNK // 2, pair_body, 0, unroll=2)
            row0 = pl.multiple_of(tok0 // 2, CHUNK // 2)
            pltpu.sync_copy(o_v, out_hbm.at[pl.ds(row0, CHUNK // 2)])
            return 0

        lax.fori_loop(0, n_chunks, chunk_body, 0)

    return gather_sum


def kernel(input, W0, W1, W2, Wm, b):
    seq, batch, _ = input.shape
    n = seq * batch
    idx = input.reshape(n, 3).astype(jnp.int32)
    i0 = idx[:, 0]
    i1 = idx[:, 1]
    i2 = idx[:, 2]
    M0 = Wm[:, 0 * DIM:1 * DIM].T
    M1 = Wm[:, 1 * DIM:2 * DIM].T
    M2 = Wm[:, 2 * DIM:3 * DIM].T
    bb = (b * (1.0 / 3.0)).reshape(1, DIM)
    pa, pb = _project(W0, W1, W2, M0, M1, M2, bb)
    out = _make_gather_sum(n)(pa, pb, i0, i1, i2)
    return out.reshape(seq, batch, DIM)


# speedup vs baseline: 3.5115x; 1.1215x over previous
import functools

import jax
import jax.numpy as jnp
from jax import lax
from jax.experimental import pallas as pl
from jax.experimental.pallas import tpu as pltpu
from jax.experimental.pallas import tpu_sc as plsc

DIM = 64
LANES = 16
NC, NS = 2, 16
NW = NC * NS
CHUNK = 128
NSLOT = 2

PROJ_BLK = 1000


def _project_kernel(w0, w1, w2, m0, m1, m2, bb, pa, pb):
    c0 = jnp.dot(w0[...], m0[...], preferred_element_type=jnp.float32)
    c1 = jnp.dot(w1[...], m1[...], preferred_element_type=jnp.float32)
    c2 = jnp.dot(w2[...], m2[...], preferred_element_type=jnp.float32)
    third = bb[...]
    pa[...] = jnp.concatenate([c0 + third, c1 + third], axis=1)
    pb[...] = jnp.concatenate([c2 + third, c2 + third], axis=1)


def _project(W0, W1, W2, M0, M1, M2, bb):
    vocab = W0.shape[0]
    assert vocab % PROJ_BLK == 0
    nblk = vocab // PROJ_BLK
    row_spec = pl.BlockSpec((PROJ_BLK, DIM), lambda i: (i, 0))
    out_spec = pl.BlockSpec((PROJ_BLK, 2 * DIM), lambda i: (i, 0))
    full_spec = pl.BlockSpec((DIM, DIM), lambda i: (0, 0))
    bias_spec = pl.BlockSpec((1, DIM), lambda i: (0, 0))
    return pl.pallas_call(
        _project_kernel,
        grid=(nblk,),
        in_specs=[row_spec, row_spec, row_spec,
                  full_spec, full_spec, full_spec, bias_spec],
        out_specs=[out_spec, out_spec],
        out_shape=[jax.ShapeDtypeStruct((vocab, 2 * DIM), jnp.float32)] * 2,
    )(W0, W1, W2, M0, M1, M2, bb)


def _make_gather_sum(n_tokens):
    n_per_w = n_tokens // NW
    n_chunks = n_per_w // CHUNK
    assert n_per_w % CHUNK == 0
    mesh = plsc.VectorSubcoreMesh(core_axis_name="c", subcore_axis_name="s")

    @functools.partial(
        pl.kernel,
        mesh=mesh,
        out_type=jax.ShapeDtypeStruct((n_tokens // 2, 2 * DIM), jnp.float32),
        scratch_types=[
            pltpu.VMEM((NSLOT, CHUNK), jnp.int32),
            pltpu.VMEM((NSLOT, CHUNK), jnp.int32),
            pltpu.VMEM((NSLOT, CHUNK), jnp.int32),
            pltpu.VMEM((NSLOT, CHUNK, 2 * DIM), jnp.float32),
            pltpu.VMEM((NSLOT, CHUNK, 2 * DIM), jnp.float32),
            pltpu.VMEM((NSLOT, CHUNK, 2 * DIM), jnp.float32),
            pltpu.VMEM((CHUNK // 2, 2 * DIM), jnp.float32),
            pltpu.SemaphoreType.DMA((NSLOT,)),
        ],
    )
    def gather_sum(pa_hbm, pb_hbm, i0_hbm, i1_hbm, i2_hbm,
                   out_hbm, i0_v, i1_v, i2_v, r0_v, r1_v, r2_v, o_v, sems):
        wid = lax.axis_index("s") * NC + lax.axis_index("c")
        base = wid * n_per_w

        def start(ch, slot):
            tok0 = pl.multiple_of(base + ch * CHUNK, CHUNK)
            toks = pl.ds(tok0, CHUNK)
            pltpu.sync_copy(i0_hbm.at[toks], i0_v.at[slot])
            pltpu.sync_copy(i1_hbm.at[toks], i1_v.at[slot])
            pltpu.sync_copy(i2_hbm.at[toks], i2_v.at[slot])
            pltpu.async_copy(
                pa_hbm.at[i0_v.at[slot]], r0_v.at[slot], sems.at[slot])
            pltpu.async_copy(
                pa_hbm.at[i1_v.at[slot]], r1_v.at[slot], sems.at[slot])
            pltpu.async_copy(
                pb_hbm.at[i2_v.at[slot]], r2_v.at[slot], sems.at[slot])

        def finish(ch, slot):
            for rv in (r0_v, r1_v, r2_v):
                pltpu.make_async_copy(
                    pa_hbm.at[i0_v.at[slot]], rv.at[slot], sems.at[slot]
                ).wait()
            r0s, r1s, r2s = r0_v.at[slot], r1_v.at[slot], r2_v.at[slot]

            def pair_body(u, _):
                for r in range(2):
                    t = 2 * u + r
                    for k in range(DIM // LANES):
                        src = pl.ds(k * LANES, LANES)
                        hi = pl.ds(DIM + k * LANES, LANES)
                        v = r0s[t, src] + r1s[t, hi] + r2s[t, src]
                        o_v[u, pl.ds(r * DIM + k * LANES, LANES)] = (
                            jnp.maximum(v, 0.0))
                return 0

            lax.fori_loop(0, CHUNK // 2, pair_body, 0, unroll=2)
            tok0 = pl.multiple_of(base + ch * CHUNK, CHUNK)
            row0 = pl.multiple_of(tok0 // 2, CHUNK // 2)
            pltpu.sync_copy(o_v, out_hbm.at[pl.ds(row0, CHUNK // 2)])

        n_pairs = n_chunks // NSLOT
        start(0, 0)

        def pair_of_chunks(g, _):
            start(NSLOT * g + 1, 1)
            finish(NSLOT * g, 0)

            @pl.when(g + 1 < n_pairs)
            def _():
                start(NSLOT * g + 2, 0)

            finish(NSLOT * g + 1, 1)
            return 0

        lax.fori_loop(0, n_pairs, pair_of_chunks, 0)

    return gather_sum


def kernel(input, W0, W1, W2, Wm, b):
    seq, batch, _ = input.shape
    n = seq * batch
    idx = input.reshape(n, 3).astype(jnp.int32)
    i0 = idx[:, 0]
    i1 = idx[:, 1]
    i2 = idx[:, 2]
    M0 = Wm[:, 0 * DIM:1 * DIM].T
    M1 = Wm[:, 1 * DIM:2 * DIM].T
    M2 = Wm[:, 2 * DIM:3 * DIM].T
    bb = (b * (1.0 / 3.0)).reshape(1, DIM)
    pa, pb = _project(W0, W1, W2, M0, M1, M2, bb)
    out = _make_gather_sum(n)(pa, pb, i0, i1, i2)
    return out.reshape(seq, batch, DIM)


# speedup vs baseline: 4.1790x; 1.1901x over previous
import functools

import jax
import jax.numpy as jnp
from jax import lax
from jax.experimental import pallas as pl
from jax.experimental.pallas import tpu as pltpu
from jax.experimental.pallas import tpu_sc as plsc

DIM = 64
LANES = 16
NC, NS = 2, 16
NW = NC * NS
CHUNK = 128
NSLOT = 2

PROJ_BLK = 2000


def _project_kernel(w0, w1, w2, m0, m1, m2, bb, pa, pb):
    c0 = jnp.dot(w0[...], m0[...], preferred_element_type=jnp.float32)
    c1 = jnp.dot(w1[...], m1[...], preferred_element_type=jnp.float32)
    c2 = jnp.dot(w2[...], m2[...], preferred_element_type=jnp.float32)
    third = bb[...]
    pa[...] = jnp.concatenate([c0 + third, c1 + third], axis=1)
    pb[...] = jnp.concatenate([c2 + third, c2 + third], axis=1)


def _project(W0, W1, W2, M0, M1, M2, bb):
    vocab = W0.shape[0]
    assert vocab % PROJ_BLK == 0
    nblk = vocab // PROJ_BLK
    row_spec = pl.BlockSpec((PROJ_BLK, DIM), lambda i: (i, 0))
    out_spec = pl.BlockSpec((PROJ_BLK, 2 * DIM), lambda i: (i, 0))
    full_spec = pl.BlockSpec((DIM, DIM), lambda i: (0, 0))
    bias_spec = pl.BlockSpec((1, DIM), lambda i: (0, 0))
    return pl.pallas_call(
        _project_kernel,
        grid=(nblk,),
        in_specs=[row_spec, row_spec, row_spec,
                  full_spec, full_spec, full_spec, bias_spec],
        out_specs=[out_spec, out_spec],
        out_shape=[jax.ShapeDtypeStruct((vocab, 2 * DIM), jnp.float32)] * 2,
    )(W0, W1, W2, M0, M1, M2, bb)


def _make_gather_sum(n_tokens):
    n_per_w = n_tokens // NW
    n_chunks = n_per_w // CHUNK
    assert n_per_w % CHUNK == 0
    mesh = plsc.VectorSubcoreMesh(core_axis_name="c", subcore_axis_name="s")

    @functools.partial(
        pl.kernel,
        mesh=mesh,
        out_type=jax.ShapeDtypeStruct((n_tokens // 2, 2 * DIM), jnp.float32),
        scratch_types=[
            pltpu.VMEM((n_per_w,), jnp.int32),
            pltpu.VMEM((n_per_w,), jnp.int32),
            pltpu.VMEM((n_per_w,), jnp.int32),
            pltpu.VMEM((NSLOT, CHUNK, 2 * DIM), jnp.float32),
            pltpu.VMEM((NSLOT, CHUNK, 2 * DIM), jnp.float32),
            pltpu.VMEM((NSLOT, CHUNK, 2 * DIM), jnp.float32),
            pltpu.VMEM((CHUNK // 2, 2 * DIM), jnp.float32),
            pltpu.SemaphoreType.DMA((NSLOT,)),
        ],
    )
    def gather_sum(pa_hbm, pb_hbm, i0_hbm, i1_hbm, i2_hbm,
                   out_hbm, i0_v, i1_v, i2_v, r0_v, r1_v, r2_v, o_v, sems):
        wid = lax.axis_index("s") * NC + lax.axis_index("c")
        base = wid * n_per_w

        toks_all = pl.ds(pl.multiple_of(base, CHUNK), n_per_w)
        pltpu.sync_copy(i0_hbm.at[toks_all], i0_v)
        pltpu.sync_copy(i1_hbm.at[toks_all], i1_v)
        pltpu.sync_copy(i2_hbm.at[toks_all], i2_v)

        def start(ch, slot):
            sl = pl.ds(pl.multiple_of(ch * CHUNK, CHUNK), CHUNK)
            pltpu.async_copy(
                pa_hbm.at[i0_v.at[sl]], r0_v.at[slot], sems.at[slot])
            pltpu.async_copy(
                pa_hbm.at[i1_v.at[sl]], r1_v.at[slot], sems.at[slot])
            pltpu.async_copy(
                pb_hbm.at[i2_v.at[sl]], r2_v.at[slot], sems.at[slot])

        def finish(ch, slot):
            for rv in (r0_v, r1_v, r2_v):
                pltpu.make_async_copy(
                    pa_hbm.at[i0_v.at[pl.ds(0, CHUNK)]], rv.at[slot],
                    sems.at[slot]).wait()
            r0s, r1s, r2s = r0_v.at[slot], r1_v.at[slot], r2_v.at[slot]

            def pair_body(u, _):
                for r in range(2):
                    t = 2 * u + r
                    for k in range(DIM // LANES):
                        src = pl.ds(k * LANES, LANES)
                        hi = pl.ds(DIM + k * LANES, LANES)
                        v = r0s[t, src] + r1s[t, hi] + r2s[t, src]
                        o_v[u, pl.ds(r * DIM + k * LANES, LANES)] = (
                            jnp.maximum(v, 0.0))
                return 0

            lax.fori_loop(0, CHUNK // 2, pair_body, 0, unroll=2)
            tok0 = pl.multiple_of(base + ch * CHUNK, CHUNK)
            row0 = pl.multiple_of(tok0 // 2, CHUNK // 2)
            pltpu.sync_copy(o_v, out_hbm.at[pl.ds(row0, CHUNK // 2)])

        n_pairs = n_chunks // NSLOT
        start(0, 0)

        def pair_of_chunks(g, _):
            start(NSLOT * g + 1, 1)
            finish(NSLOT * g, 0)

            @pl.when(g + 1 < n_pairs)
            def _():
                start(NSLOT * g + 2, 0)

            finish(NSLOT * g + 1, 1)
            return 0

        lax.fori_loop(0, n_pairs, pair_of_chunks, 0)

    return gather_sum


def kernel(input, W0, W1, W2, Wm, b):
    seq, batch, _ = input.shape
    n = seq * batch
    idx = input.reshape(n, 3).astype(jnp.int32)
    i0 = idx[:, 0]
    i1 = idx[:, 1]
    i2 = idx[:, 2]
    M0 = Wm[:, 0 * DIM:1 * DIM].T
    M1 = Wm[:, 1 * DIM:2 * DIM].T
    M2 = Wm[:, 2 * DIM:3 * DIM].T
    bb = (b * (1.0 / 3.0)).reshape(1, DIM)
    pa, pb = _project(W0, W1, W2, M0, M1, M2, bb)
    out = _make_gather_sum(n)(pa, pb, i0, i1, i2)
    return out.reshape(seq, batch, DIM)
